# Initial kernel scaffold; baseline (speedup 1.0000x reference)
import jax
import jax.numpy as jnp
from jax.experimental import pallas as pl


def kernel(token_ids, weight):
    raise NotImplementedError("write your pallas kernel here")


# speedup vs baseline: 1.4014x; 1.4014x over previous
import functools

import jax
import jax.numpy as jnp
from jax import lax
from jax.experimental import pallas as pl
from jax.experimental.pallas import tpu as pltpu
from jax.experimental.pallas import tpu_sc as plsc

_NUM_WORKERS = 32
_CHUNK = 128
_DIM = 32


@functools.partial(jax.jit, static_argnums=(2, 3))
def _emb_lookup(idx, weight, n_per_w, n_chunks):
    mesh = plsc.VectorSubcoreMesh(core_axis_name="c", subcore_axis_name="s")

    @functools.partial(
        pl.kernel,
        out_type=jax.ShapeDtypeStruct((_NUM_WORKERS * n_per_w, _DIM), jnp.float32),
        mesh=mesh,
        scratch_types=[
            pltpu.VMEM((n_chunks, _CHUNK), jnp.int32),
            pltpu.VMEM((_CHUNK, _DIM), jnp.float32),
            pltpu.SemaphoreType.DMA,
        ],
        compiler_params=pltpu.CompilerParams(use_tc_tiling_on_sc=False),
    )
    def body(idx_hbm, table_hbm, out_hbm, idx_v, rows_v, sem):
        wid = lax.axis_index("s") * 2 + lax.axis_index("c")
        base = wid * n_per_w
        pltpu.sync_copy(idx_hbm.at[wid], idx_v)

        def step(j, carry):
            pltpu.async_copy(table_hbm.at[idx_v.at[j]], rows_v, sem).wait()
            pltpu.sync_copy(rows_v, out_hbm.at[pl.ds(base + j * _CHUNK, _CHUNK)])
            return carry

        lax.fori_loop(0, n_chunks, step, 0)

    return body(idx, weight)


def kernel(token_ids, weight):
    n_total = token_ids.shape[0] * token_ids.shape[1]
    n_per_w = n_total // _NUM_WORKERS
    n_chunks = n_per_w // _CHUNK
    idx = token_ids.astype(jnp.int32).reshape(_NUM_WORKERS, n_chunks, _CHUNK)
    out = _emb_lookup(idx, weight, n_per_w, n_chunks)
    return out.reshape(*token_ids.shape, _DIM)


# speedup vs baseline: 1.5126x; 1.0794x over previous
import functools

import jax
import jax.numpy as jnp
from jax import lax
from jax.experimental import pallas as pl
from jax.experimental.pallas import tpu as pltpu
from jax.experimental.pallas import tpu_sc as plsc

_NUM_WORKERS = 32
_CHUNK = 128
_NBUF = 8
_DIM = 32


@functools.partial(jax.jit, static_argnums=(2,))
def _emb_lookup(idx, weight, n_chunks):
    n_groups = n_chunks // _NBUF
    mesh = plsc.VectorSubcoreMesh(core_axis_name="c", subcore_axis_name="s")

    @functools.partial(
        pl.kernel,
        out_type=jax.ShapeDtypeStruct(
            (_NUM_WORKERS * n_chunks, _CHUNK, _DIM), jnp.float32
        ),
        mesh=mesh,
        scratch_types=[
            pltpu.VMEM((n_chunks, _CHUNK), jnp.int32),
            pltpu.VMEM((2, _NBUF, _CHUNK, _DIM), jnp.float32),
            pltpu.SemaphoreType.DMA((2,)),
            pltpu.SemaphoreType.DMA((2,)),
        ],
        compiler_params=pltpu.CompilerParams(use_tc_tiling_on_sc=False),
    )
    def body(idx_hbm, table_hbm, out_hbm, idx_v, sbuf, gsem, ssem):
        wid = lax.axis_index("s") * 2 + lax.axis_index("c")
        base = wid * n_chunks
        pltpu.sync_copy(idx_hbm.at[wid], idx_v)

        def launch_gathers(g, sb):
            for b in range(_NBUF):
                pltpu.async_copy(
                    table_hbm.at[idx_v.at[g * _NBUF + b]],
                    sbuf.at[sb, b],
                    gsem.at[sb],
                )

        def wait_gathers(g, sb):
            for b in range(_NBUF):
                pltpu.make_async_copy(
                    table_hbm.at[idx_v.at[g * _NBUF + b]],
                    sbuf.at[sb, b],
                    gsem.at[sb],
                ).wait()

        def scatter_desc(g, sb):
            return pltpu.make_async_copy(
                sbuf.at[sb],
                out_hbm.at[pl.ds(base + g * _NBUF, _NBUF)],
                ssem.at[sb],
            )

        launch_gathers(0, 0)

        def group(g, carry):
            sb = lax.rem(g, 2)
            wait_gathers(g, sb)
            scatter_desc(g, sb).start()

            @pl.when(g + 1 < n_groups)
            def _():
                @pl.when(g >= 1)
                def _():
                    scatter_desc(g - 1, 1 - sb).wait()

                launch_gathers(g + 1, 1 - sb)

            return carry

        lax.fori_loop(0, n_groups, group, 0)
        scatter_desc(n_groups - 2, n_groups % 2).wait()
        scatter_desc(n_groups - 1, (n_groups - 1) % 2).wait()

    return body(idx, weight)


def kernel(token_ids, weight):
    n_total = token_ids.shape[0] * token_ids.shape[1]
    n_chunks = n_total // (_NUM_WORKERS * _CHUNK)
    idx = token_ids.astype(jnp.int32).reshape(_NUM_WORKERS, n_chunks, _CHUNK)
    out = _emb_lookup(idx, weight, n_chunks)
    return out.reshape(*token_ids.shape, _DIM)
